# baseline (speedup 1.0000x reference)
import jax
import jax.numpy as jnp
from jax import lax
from jax.experimental import pallas as pl
from jax.experimental.pallas import tpu as pltpu
from jax.experimental.pallas import tpu_sc as plsc

N = 50000
E = 800000
IN_C = 100
HID = 64
OUT_C = 18

G = 250
EROWS = E // G
CH = 20
N_PAD = 50176
ROWS_PER_TILE = N_PAD // 16
BN = 2000
GRID = N // BN

_mesh = plsc.VectorSubcoreMesh(core_axis_name="c", subcore_axis_name="s")
_f32 = jnp.float32
_params = pltpu.CompilerParams(use_tc_tiling_on_sc=False)


@pl.kernel(
    out_type=jax.ShapeDtypeStruct((2, N_PAD, 8), _f32),
    mesh=_mesh,
    compiler_params=_params,
    scratch_types=[
        pltpu.VMEM((CH, G), jnp.int32),
        pltpu.VMEM((G, 8), _f32),
        pltpu.VMEM_SHARED((N_PAD, 8), _f32),
        pltpu.SemaphoreType.DMA,
    ],
)
def _sc_deg(dst_hbm, zeros_hbm, ones_hbm, out_hbm, idx_v, ones_v, table, sem):
    cid = lax.axis_index("c")
    sid = lax.axis_index("s")
    r0 = sid * ROWS_PER_TILE
    pltpu.sync_copy(zeros_hbm, table.at[pl.ds(r0, ROWS_PER_TILE)])
    pltpu.sync_copy(ones_hbm, ones_v)
    plsc.subcore_barrier()

    base = cid * (EROWS // 2) + sid * (EROWS // 32)

    @pl.loop(0, EROWS // 32 // CH)
    def _chunks(ch):
        pltpu.sync_copy(dst_hbm.at[pl.ds(base + ch * CH, CH)], idx_v)

        @pl.loop(0, CH)
        def _groups(g):
            pltpu.sync_copy(ones_v, table.at[idx_v.at[g]], add=True)

    plsc.subcore_barrier()
    pltpu.sync_copy(table.at[pl.ds(r0, ROWS_PER_TILE)],
                    out_hbm.at[cid, pl.ds(r0, ROWS_PER_TILE)])


def _make_agg(w):
    nch = (EROWS // 16) // CH

    def body(hp_hbm, src_hbm, srcN_hbm, dst_hbm, zeros_hbm, out_hbm,
             src_v, dst_v, bufs, acc, sg0, sg1, ss0, ss1):
        cid = lax.axis_index("c")
        sid = lax.axis_index("s")
        r0 = sid * ROWS_PER_TILE
        pltpu.sync_copy(zeros_hbm, acc.at[pl.ds(r0, ROWS_PER_TILE)])
        plsc.subcore_barrier()
        base = sid * (EROWS // 16)

        def drain(ss, k):
            pltpu.make_async_copy(hp_hbm.at[pl.ds(0, G)], bufs.at[k], ss).wait()

        @pl.loop(0, nch)
        def _chunk(ch):
            @pl.when(ch > 0)
            def _():
                drain(ss0, 0)
                drain(ss1, 1)

            row0 = base + ch * CH

            @pl.when(cid == 0)
            def _():
                pltpu.sync_copy(src_hbm.at[pl.ds(row0, CH)], src_v)

            @pl.when(cid == 1)
            def _():
                pltpu.sync_copy(srcN_hbm.at[pl.ds(row0, CH)], src_v)

            pltpu.sync_copy(dst_hbm.at[pl.ds(row0, CH)], dst_v)

            @pl.loop(0, CH // 2)
            def _pipe(p):
                @pl.when(p > 0)
                def _():
                    drain(ss0, 0)
                    drain(ss1, 1)

                d0 = pltpu.async_copy(hp_hbm.at[src_v.at[2 * p]],
                                      bufs.at[0], sg0)
                d1 = pltpu.async_copy(hp_hbm.at[src_v.at[2 * p + 1]],
                                      bufs.at[1], sg1)
                d0.wait()
                pltpu.async_copy(bufs.at[0], acc.at[dst_v.at[2 * p]], ss0,
                                 add=True)
                d1.wait()
                pltpu.async_copy(bufs.at[1], acc.at[dst_v.at[2 * p + 1]], ss1,
                                 add=True)

        drain(ss0, 0)
        drain(ss1, 1)
        plsc.subcore_barrier()
        pltpu.sync_copy(acc.at[pl.ds(r0, ROWS_PER_TILE)],
                        out_hbm.at[cid, pl.ds(r0, ROWS_PER_TILE)])

    return pl.kernel(
        body,
        out_type=jax.ShapeDtypeStruct((2, N_PAD, w), _f32),
        mesh=_mesh,
        compiler_params=_params,
        scratch_types=[
            pltpu.VMEM((CH, G), jnp.int32),
            pltpu.VMEM((CH, G), jnp.int32),
            pltpu.VMEM((2, G, w), _f32),
            pltpu.VMEM_SHARED((N_PAD, w), _f32),
            pltpu.SemaphoreType.DMA,
            pltpu.SemaphoreType.DMA,
            pltpu.SemaphoreType.DMA,
            pltpu.SemaphoreType.DMA,
        ],
    )


_sc_agg64 = _make_agg(32)
_sc_agg32 = _make_agg(16)


def _tc1_body(x_ref, w_ref, degp_ref, h_ref, hp_ref, dinv_ref):
    deg = degp_ref[0, :, 0:1] + degp_ref[1, :, 0:1] + 1.0
    dinv = lax.rsqrt(deg)
    dinv_ref[...] = dinv
    h = jnp.dot(x_ref[...], w_ref[...], preferred_element_type=_f32)
    h_ref[...] = h
    hp = h * dinv
    hp_ref[0, :, :] = hp[:, :32]
    hp_ref[1, :, :] = hp[:, 32:]


def _tc_mid_body(acc_ref, hprev_ref, dinv_ref, b_ref, w_ref, h_ref, hp_ref):
    dinv = dinv_ref[...]
    accf = jnp.concatenate([acc_ref[0, :, :], acc_ref[1, :, :]], axis=1)
    g = jax.nn.relu(accf * dinv + hprev_ref[...] * (dinv * dinv) + b_ref[...])
    h = jnp.dot(g, w_ref[...], preferred_element_type=_f32)
    h_ref[...] = h
    hp = h * dinv
    hp_ref[0, :, :] = hp[:, :32]
    hp_ref[1, :, :] = hp[:, 32:]


def _tc4_body(acc_ref, hprev_ref, dinv_ref, b_ref, w_ref, h_ref, hp_ref):
    dinv = dinv_ref[...]
    accf = jnp.concatenate([acc_ref[0, :, :], acc_ref[1, :, :]], axis=1)
    g = jax.nn.relu(accf * dinv + hprev_ref[...] * (dinv * dinv) + b_ref[...])
    h = jnp.dot(g, w_ref[...], preferred_element_type=_f32)
    h_ref[...] = h
    hp = h * dinv
    hp_ref[0, :, :] = hp[:, :16]
    hp_ref[1, :, :] = hp[:, 16:]


def _tc5_body(acc_ref, hprev_ref, dinv_ref, b_ref, out_ref):
    dinv = dinv_ref[...]
    accf = jnp.concatenate([acc_ref[0, :, :], acc_ref[1, :, :]], axis=1)
    out_ref[...] = accf * dinv + hprev_ref[...] * (dinv * dinv) + b_ref[...]


def _row_spec(c):
    return pl.BlockSpec((BN, c), lambda i: (i, 0))


def _split_spec(c):
    return pl.BlockSpec((2, BN, c), lambda i: (0, i, 0))


def _full_spec(r, c):
    return pl.BlockSpec((r, c), lambda i: (0, 0))


def kernel(x, edge_index, W1, b1, W2, b2, W3, b3, W4, b4):
    src = edge_index[0].reshape(EROWS, G)
    srcN = (edge_index[0] + N).reshape(EROWS, G)
    dst = edge_index[1].reshape(EROWS, G)
    zeros32 = jnp.zeros((ROWS_PER_TILE, 32), _f32)
    zeros16 = jnp.zeros((ROWS_PER_TILE, 16), _f32)
    zeros8 = jnp.zeros((ROWS_PER_TILE, 8), _f32)
    ones8 = jnp.ones((G, 8), _f32)
    W4p = jnp.pad(W4, ((0, 0), (0, 32 - OUT_C)))
    b4p = jnp.pad(b4, (0, 32 - OUT_C))

    degp = _sc_deg(dst, zeros8, ones8)

    tc1 = pl.pallas_call(
        _tc1_body,
        grid=(GRID,),
        in_specs=[_row_spec(IN_C), _full_spec(IN_C, HID), _split_spec(8)],
        out_specs=[_row_spec(HID), _split_spec(32), _row_spec(1)],
        out_shape=[jax.ShapeDtypeStruct((N, HID), _f32),
                   jax.ShapeDtypeStruct((2, N, 32), _f32),
                   jax.ShapeDtypeStruct((N, 1), _f32)],
    )
    h1, hp1, dinv = tc1(x, W1, degp)

    tc_mid = pl.pallas_call(
        _tc_mid_body,
        grid=(GRID,),
        in_specs=[_split_spec(32), _row_spec(HID), _row_spec(1),
                  _full_spec(1, HID), _full_spec(HID, HID)],
        out_specs=[_row_spec(HID), _split_spec(32)],
        out_shape=[jax.ShapeDtypeStruct((N, HID), _f32),
                   jax.ShapeDtypeStruct((2, N, 32), _f32)],
    )

    acc1 = _sc_agg64(hp1.reshape(2 * N, 32), src, srcN, dst, zeros32)
    h2, hp2 = tc_mid(acc1, h1, dinv, b1[None, :], W2)

    acc2 = _sc_agg64(hp2.reshape(2 * N, 32), src, srcN, dst, zeros32)
    h3, hp3 = tc_mid(acc2, h2, dinv, b2[None, :], W3)

    acc3 = _sc_agg64(hp3.reshape(2 * N, 32), src, srcN, dst, zeros32)
    tc4 = pl.pallas_call(
        _tc4_body,
        grid=(GRID,),
        in_specs=[_split_spec(32), _row_spec(HID), _row_spec(1),
                  _full_spec(1, HID), _full_spec(HID, 32)],
        out_specs=[_row_spec(32), _split_spec(16)],
        out_shape=[jax.ShapeDtypeStruct((N, 32), _f32),
                   jax.ShapeDtypeStruct((2, N, 16), _f32)],
    )
    h4, hp4 = tc4(acc3, h3, dinv, b3[None, :], W4p)

    acc4 = _sc_agg32(hp4.reshape(2 * N, 16), src, srcN, dst, zeros16)
    tc5 = pl.pallas_call(
        _tc5_body,
        grid=(GRID,),
        in_specs=[_split_spec(16), _row_spec(32), _row_spec(1),
                  _full_spec(1, 32)],
        out_specs=_row_spec(32),
        out_shape=jax.ShapeDtypeStruct((N, 32), _f32),
    )
    out = tc5(acc4, h4, dinv, b4p[None, :])
    return out[:, :OUT_C]


# speedup vs baseline: 1.0259x; 1.0259x over previous
import jax
import jax.numpy as jnp
from jax import lax
from jax.experimental import pallas as pl
from jax.experimental.pallas import tpu as pltpu
from jax.experimental.pallas import tpu_sc as plsc

N = 50000
E = 800000
IN_C = 100
HID = 64
OUT_C = 18

G = 250
EROWS = E // G
CH = 20
N_PAD = 50176
ROWS_PER_TILE = N_PAD // 16
BN = 2000
GRID = N // BN

_mesh = plsc.VectorSubcoreMesh(core_axis_name="c", subcore_axis_name="s")
_f32 = jnp.float32
_params = pltpu.CompilerParams(use_tc_tiling_on_sc=False)


@pl.kernel(
    out_type=jax.ShapeDtypeStruct((2, N_PAD, 8), _f32),
    mesh=_mesh,
    compiler_params=_params,
    scratch_types=[
        pltpu.VMEM((CH, G), jnp.int32),
        pltpu.VMEM((G, 8), _f32),
        pltpu.VMEM_SHARED((N_PAD, 8), _f32),
        pltpu.SemaphoreType.DMA,
    ],
)
def _sc_deg(dst_hbm, zeros_hbm, ones_hbm, out_hbm, idx_v, ones_v, table, sem):
    cid = lax.axis_index("c")
    sid = lax.axis_index("s")
    r0 = sid * ROWS_PER_TILE
    pltpu.sync_copy(zeros_hbm, table.at[pl.ds(r0, ROWS_PER_TILE)])
    pltpu.sync_copy(ones_hbm, ones_v)
    plsc.subcore_barrier()

    base = cid * (EROWS // 2) + sid * (EROWS // 32)

    @pl.loop(0, EROWS // 32 // CH)
    def _chunks(ch):
        pltpu.sync_copy(dst_hbm.at[pl.ds(base + ch * CH, CH)], idx_v)

        @pl.loop(0, CH)
        def _groups(g):
            pltpu.sync_copy(ones_v, table.at[idx_v.at[g]], add=True)

    plsc.subcore_barrier()
    pltpu.sync_copy(table.at[pl.ds(r0, ROWS_PER_TILE)],
                    out_hbm.at[cid, pl.ds(r0, ROWS_PER_TILE)])


def _make_agg(w):
    nch = (EROWS // 16) // CH

    def body(hp_hbm, src_hbm, dst_hbm, zeros_hbm, out_hbm,
             src_v, dst_v, bufs, acc, sg0, sg1, ss0, ss1):
        cid = lax.axis_index("c")
        sid = lax.axis_index("s")
        r0 = sid * ROWS_PER_TILE
        pltpu.sync_copy(zeros_hbm, acc.at[pl.ds(r0, ROWS_PER_TILE)])
        plsc.subcore_barrier()
        base = sid * (EROWS // 16)
        hp_c = hp_hbm.at[cid]

        def drain(ss, k):
            pltpu.make_async_copy(hp_c.at[pl.ds(0, G)], bufs.at[k], ss).wait()

        @pl.loop(0, nch)
        def _chunk(ch):
            @pl.when(ch > 0)
            def _():
                drain(ss0, 0)
                drain(ss1, 1)

            row0 = base + ch * CH
            pltpu.sync_copy(src_hbm.at[pl.ds(row0, CH)], src_v)
            pltpu.sync_copy(dst_hbm.at[pl.ds(row0, CH)], dst_v)

            @pl.loop(0, CH // 2)
            def _pipe(p):
                @pl.when(p > 0)
                def _():
                    drain(ss0, 0)
                    drain(ss1, 1)

                d0 = pltpu.async_copy(hp_c.at[src_v.at[2 * p]],
                                      bufs.at[0], sg0)
                d1 = pltpu.async_copy(hp_c.at[src_v.at[2 * p + 1]],
                                      bufs.at[1], sg1)
                d0.wait()
                pltpu.async_copy(bufs.at[0], acc.at[dst_v.at[2 * p]], ss0,
                                 add=True)
                d1.wait()
                pltpu.async_copy(bufs.at[1], acc.at[dst_v.at[2 * p + 1]], ss1,
                                 add=True)

        drain(ss0, 0)
        drain(ss1, 1)
        plsc.subcore_barrier()
        pltpu.sync_copy(acc.at[pl.ds(r0, ROWS_PER_TILE)],
                        out_hbm.at[pl.ds(r0, ROWS_PER_TILE),
                                   pl.ds(cid * w, w)])

    return pl.kernel(
        body,
        out_type=jax.ShapeDtypeStruct((N_PAD, 2 * w), _f32),
        mesh=_mesh,
        compiler_params=_params,
        scratch_types=[
            pltpu.VMEM((CH, G), jnp.int32),
            pltpu.VMEM((CH, G), jnp.int32),
            pltpu.VMEM((2, G, w), _f32),
            pltpu.VMEM_SHARED((N_PAD, w), _f32),
            pltpu.SemaphoreType.DMA,
            pltpu.SemaphoreType.DMA,
            pltpu.SemaphoreType.DMA,
            pltpu.SemaphoreType.DMA,
        ],
    )


_sc_agg64 = _make_agg(32)
_sc_agg32 = _make_agg(16)


def _tc1_body(x_ref, w_ref, degp_ref, h_ref, hp_ref, dinv_ref):
    deg = degp_ref[0, :, 0:1] + degp_ref[1, :, 0:1] + 1.0
    dinv = lax.rsqrt(deg)
    dinv_ref[...] = dinv
    h = jnp.dot(x_ref[...], w_ref[...], preferred_element_type=_f32)
    h_ref[...] = h
    hp = h * dinv
    hp_ref[0, :, :] = hp[:, :32]
    hp_ref[1, :, :] = hp[:, 32:]


def _tc_mid_body(acc_ref, hprev_ref, dinv_ref, b_ref, w_ref, h_ref, hp_ref):
    dinv = dinv_ref[...]
    g = jax.nn.relu(acc_ref[...] * dinv + hprev_ref[...] * (dinv * dinv)
                    + b_ref[...])
    h = jnp.dot(g, w_ref[...], preferred_element_type=_f32)
    h_ref[...] = h
    hp = h * dinv
    hp_ref[0, :, :] = hp[:, :32]
    hp_ref[1, :, :] = hp[:, 32:]


def _tc4_body(acc_ref, hprev_ref, dinv_ref, b_ref, w_ref, h_ref, hp_ref):
    dinv = dinv_ref[...]
    g = jax.nn.relu(acc_ref[...] * dinv + hprev_ref[...] * (dinv * dinv)
                    + b_ref[...])
    h = jnp.dot(g, w_ref[...], preferred_element_type=_f32)
    h_ref[...] = h
    hp = h * dinv
    hp_ref[0, :, :] = hp[:, :16]
    hp_ref[1, :, :] = hp[:, 16:]


def _tc5_body(acc_ref, hprev_ref, dinv_ref, b_ref, out_ref):
    dinv = dinv_ref[...]
    out_ref[...] = (acc_ref[...] * dinv + hprev_ref[...] * (dinv * dinv)
                    + b_ref[...])


def _row_spec(c):
    return pl.BlockSpec((BN, c), lambda i: (i, 0))


def _split_spec(c):
    return pl.BlockSpec((2, BN, c), lambda i: (0, i, 0))


def _full_spec(r, c):
    return pl.BlockSpec((r, c), lambda i: (0, 0))


def kernel(x, edge_index, W1, b1, W2, b2, W3, b3, W4, b4):
    src = edge_index[0].reshape(EROWS, G)
    dst = edge_index[1].reshape(EROWS, G)
    zeros32 = jnp.zeros((ROWS_PER_TILE, 32), _f32)
    zeros16 = jnp.zeros((ROWS_PER_TILE, 16), _f32)
    zeros8 = jnp.zeros((ROWS_PER_TILE, 8), _f32)
    ones8 = jnp.ones((G, 8), _f32)
    W4p = jnp.pad(W4, ((0, 0), (0, 32 - OUT_C)))
    b4p = jnp.pad(b4, (0, 32 - OUT_C))

    degp = _sc_deg(dst, zeros8, ones8)

    tc1 = pl.pallas_call(
        _tc1_body,
        grid=(GRID,),
        in_specs=[_row_spec(IN_C), _full_spec(IN_C, HID), _split_spec(8)],
        out_specs=[_row_spec(HID), _split_spec(32), _row_spec(1)],
        out_shape=[jax.ShapeDtypeStruct((N, HID), _f32),
                   jax.ShapeDtypeStruct((2, N, 32), _f32),
                   jax.ShapeDtypeStruct((N, 1), _f32)],
    )
    h1, hp1, dinv = tc1(x, W1, degp)

    tc_mid = pl.pallas_call(
        _tc_mid_body,
        grid=(GRID,),
        in_specs=[_row_spec(HID), _row_spec(HID), _row_spec(1),
                  _full_spec(1, HID), _full_spec(HID, HID)],
        out_specs=[_row_spec(HID), _split_spec(32)],
        out_shape=[jax.ShapeDtypeStruct((N, HID), _f32),
                   jax.ShapeDtypeStruct((2, N, 32), _f32)],
    )

    acc1 = _sc_agg64(hp1, src, dst, zeros32)
    h2, hp2 = tc_mid(acc1, h1, dinv, b1[None, :], W2)

    acc2 = _sc_agg64(hp2, src, dst, zeros32)
    h3, hp3 = tc_mid(acc2, h2, dinv, b2[None, :], W3)

    acc3 = _sc_agg64(hp3, src, dst, zeros32)
    tc4 = pl.pallas_call(
        _tc4_body,
        grid=(GRID,),
        in_specs=[_row_spec(HID), _row_spec(HID), _row_spec(1),
                  _full_spec(1, HID), _full_spec(HID, 32)],
        out_specs=[_row_spec(32), _split_spec(16)],
        out_shape=[jax.ShapeDtypeStruct((N, 32), _f32),
                   jax.ShapeDtypeStruct((2, N, 16), _f32)],
    )
    h4, hp4 = tc4(acc3, h3, dinv, b3[None, :], W4p)

    acc4 = _sc_agg32(hp4, src, dst, zeros16)
    tc5 = pl.pallas_call(
        _tc5_body,
        grid=(GRID,),
        in_specs=[_row_spec(32), _row_spec(32), _row_spec(1),
                  _full_spec(1, 32)],
        out_specs=_row_spec(32),
        out_shape=jax.ShapeDtypeStruct((N, 32), _f32),
    )
    out = tc5(acc4, h4, dinv, b4p[None, :])
    return out[:, :OUT_C]


# speedup vs baseline: 1.0583x; 1.0315x over previous
import jax
import jax.numpy as jnp
from jax import lax
from jax.experimental import pallas as pl
from jax.experimental.pallas import tpu as pltpu
from jax.experimental.pallas import tpu_sc as plsc

N = 50000
E = 800000
IN_C = 100
HID = 64
OUT_C = 18

G = 250
EROWS = E // G
CH = 20
G32 = 500
EROWS32 = E // G32
CH32 = 10
N_PAD = 50176
ROWS_PER_TILE = N_PAD // 16
BN = 5000
GRID = N // BN

_mesh = plsc.VectorSubcoreMesh(core_axis_name="c", subcore_axis_name="s")
_f32 = jnp.float32
_params = pltpu.CompilerParams(use_tc_tiling_on_sc=False)


@pl.kernel(
    out_type=jax.ShapeDtypeStruct((2, N_PAD, 8), _f32),
    mesh=_mesh,
    compiler_params=_params,
    scratch_types=[
        pltpu.VMEM((CH, G), jnp.int32),
        pltpu.VMEM((G, 8), _f32),
        pltpu.VMEM_SHARED((N_PAD, 8), _f32),
        pltpu.SemaphoreType.DMA,
    ],
)
def _sc_deg(dst_hbm, zeros_hbm, ones_hbm, out_hbm, idx_v, ones_v, table, sem):
    cid = lax.axis_index("c")
    sid = lax.axis_index("s")
    r0 = sid * ROWS_PER_TILE
    pltpu.sync_copy(zeros_hbm, table.at[pl.ds(r0, ROWS_PER_TILE)])
    pltpu.sync_copy(ones_hbm, ones_v)
    plsc.subcore_barrier()

    base = cid * (EROWS // 2) + sid * (EROWS // 32)

    @pl.loop(0, EROWS // 32 // CH)
    def _chunks(ch):
        pltpu.sync_copy(dst_hbm.at[pl.ds(base + ch * CH, CH)], idx_v)

        @pl.loop(0, CH)
        def _groups(g):
            pltpu.sync_copy(ones_v, table.at[idx_v.at[g]], add=True)

    plsc.subcore_barrier()
    pltpu.sync_copy(table.at[pl.ds(r0, ROWS_PER_TILE)],
                    out_hbm.at[cid, pl.ds(r0, ROWS_PER_TILE)])


def _make_agg(w, g_, ch_, erows_):
    nch = (erows_ // 16) // ch_

    def body(hp_hbm, src_hbm, dst_hbm, zeros_hbm, out_hbm,
             src_v, dst_v, bufs, acc, sg0, sg1, ss0, ss1):
        cid = lax.axis_index("c")
        sid = lax.axis_index("s")
        r0 = sid * ROWS_PER_TILE
        pltpu.sync_copy(zeros_hbm, acc.at[pl.ds(r0, ROWS_PER_TILE)])
        plsc.subcore_barrier()
        base = sid * (erows_ // 16)
        hp_c = hp_hbm.at[cid]

        def drain(ss, k):
            pltpu.make_async_copy(hp_c.at[pl.ds(0, g_)], bufs.at[k], ss).wait()

        @pl.loop(0, nch)
        def _chunk(ch):
            @pl.when(ch > 0)
            def _():
                drain(ss0, 0)
                drain(ss1, 1)

            row0 = base + ch * ch_
            pltpu.sync_copy(src_hbm.at[pl.ds(row0, ch_)], src_v)
            pltpu.sync_copy(dst_hbm.at[pl.ds(row0, ch_)], dst_v)

            @pl.loop(0, ch_ // 2)
            def _pipe(p):
                @pl.when(p > 0)
                def _():
                    drain(ss0, 0)
                    drain(ss1, 1)

                d0 = pltpu.async_copy(hp_c.at[src_v.at[2 * p]],
                                      bufs.at[0], sg0)
                d1 = pltpu.async_copy(hp_c.at[src_v.at[2 * p + 1]],
                                      bufs.at[1], sg1)
                d0.wait()
                pltpu.async_copy(bufs.at[0], acc.at[dst_v.at[2 * p]], ss0,
                                 add=True)
                d1.wait()
                pltpu.async_copy(bufs.at[1], acc.at[dst_v.at[2 * p + 1]], ss1,
                                 add=True)

        drain(ss0, 0)
        drain(ss1, 1)
        plsc.subcore_barrier()
        pltpu.sync_copy(acc.at[pl.ds(r0, ROWS_PER_TILE)],
                        out_hbm.at[pl.ds(r0, ROWS_PER_TILE),
                                   pl.ds(cid * w, w)])

    return pl.kernel(
        body,
        out_type=jax.ShapeDtypeStruct((N_PAD, 2 * w), _f32),
        mesh=_mesh,
        compiler_params=_params,
        scratch_types=[
            pltpu.VMEM((ch_, g_), jnp.int32),
            pltpu.VMEM((ch_, g_), jnp.int32),
            pltpu.VMEM((2, g_, w), _f32),
            pltpu.VMEM_SHARED((N_PAD, w), _f32),
            pltpu.SemaphoreType.DMA,
            pltpu.SemaphoreType.DMA,
            pltpu.SemaphoreType.DMA,
            pltpu.SemaphoreType.DMA,
        ],
    )


_sc_agg64 = _make_agg(32, G, CH, EROWS)
_sc_agg32 = _make_agg(16, G32, CH32, EROWS32)


def _tc1_body(x_ref, w_ref, degp_ref, hp_ref, dinv_ref):
    deg = degp_ref[0, :, 0:1] + degp_ref[1, :, 0:1] + 1.0
    dinv = lax.rsqrt(deg)
    dinv_ref[...] = dinv
    h = jnp.dot(x_ref[...], w_ref[...], preferred_element_type=_f32)
    hp = h * dinv
    hp_ref[0, :, :] = hp[:, :32]
    hp_ref[1, :, :] = hp[:, 32:]


def _tc_mid_body(acc_ref, hpprev_ref, dinv_ref, b_ref, w_ref, hp_ref):
    dinv = dinv_ref[...]
    hpf = jnp.concatenate([hpprev_ref[0, :, :], hpprev_ref[1, :, :]], axis=1)
    g = jax.nn.relu((acc_ref[...] + hpf) * dinv + b_ref[...])
    h = jnp.dot(g, w_ref[...], preferred_element_type=_f32)
    hp = h * dinv
    hp_ref[0, :, :] = hp[:, :32]
    hp_ref[1, :, :] = hp[:, 32:]


def _tc4_body(acc_ref, hpprev_ref, dinv_ref, b_ref, w_ref, hp_ref):
    dinv = dinv_ref[...]
    hpf = jnp.concatenate([hpprev_ref[0, :, :], hpprev_ref[1, :, :]], axis=1)
    g = jax.nn.relu((acc_ref[...] + hpf) * dinv + b_ref[...])
    h = jnp.dot(g, w_ref[...], preferred_element_type=_f32)
    hp = h * dinv
    hp_ref[0, :, :] = hp[:, :16]
    hp_ref[1, :, :] = hp[:, 16:]


def _tc5_body(acc_ref, hpprev_ref, dinv_ref, b_ref, out_ref):
    dinv = dinv_ref[...]
    hpf = jnp.concatenate([hpprev_ref[0, :, :], hpprev_ref[1, :, :]], axis=1)
    out_ref[...] = (acc_ref[...] + hpf) * dinv + b_ref[...]


def _row_spec(c):
    return pl.BlockSpec((BN, c), lambda i: (i, 0))


def _split_spec(c):
    return pl.BlockSpec((2, BN, c), lambda i: (0, i, 0))


def _full_spec(r, c):
    return pl.BlockSpec((r, c), lambda i: (0, 0))


def kernel(x, edge_index, W1, b1, W2, b2, W3, b3, W4, b4):
    src = edge_index[0].reshape(EROWS, G)
    dst = edge_index[1].reshape(EROWS, G)
    src500 = edge_index[0].reshape(EROWS32, G32)
    dst500 = edge_index[1].reshape(EROWS32, G32)
    zeros32 = jnp.zeros((ROWS_PER_TILE, 32), _f32)
    zeros16 = jnp.zeros((ROWS_PER_TILE, 16), _f32)
    zeros8 = jnp.zeros((ROWS_PER_TILE, 8), _f32)
    ones8 = jnp.ones((G, 8), _f32)
    W4p = jnp.pad(W4, ((0, 0), (0, 32 - OUT_C)))
    b4p = jnp.pad(b4, (0, 32 - OUT_C))

    degp = _sc_deg(dst, zeros8, ones8)

    tc1 = pl.pallas_call(
        _tc1_body,
        grid=(GRID,),
        in_specs=[_row_spec(IN_C), _full_spec(IN_C, HID), _split_spec(8)],
        out_specs=[_split_spec(32), _row_spec(1)],
        out_shape=[jax.ShapeDtypeStruct((2, N, 32), _f32),
                   jax.ShapeDtypeStruct((N, 1), _f32)],
    )
    hp1, dinv = tc1(x, W1, degp)

    tc_mid = pl.pallas_call(
        _tc_mid_body,
        grid=(GRID,),
        in_specs=[_row_spec(HID), _split_spec(32), _row_spec(1),
                  _full_spec(1, HID), _full_spec(HID, HID)],
        out_specs=_split_spec(32),
        out_shape=jax.ShapeDtypeStruct((2, N, 32), _f32),
    )

    acc1 = _sc_agg64(hp1, src, dst, zeros32)
    hp2 = tc_mid(acc1, hp1, dinv, b1[None, :], W2)

    acc2 = _sc_agg64(hp2, src, dst, zeros32)
    hp3 = tc_mid(acc2, hp2, dinv, b2[None, :], W3)

    acc3 = _sc_agg64(hp3, src, dst, zeros32)
    tc4 = pl.pallas_call(
        _tc4_body,
        grid=(GRID,),
        in_specs=[_row_spec(HID), _split_spec(32), _row_spec(1),
                  _full_spec(1, HID), _full_spec(HID, 32)],
        out_specs=_split_spec(16),
        out_shape=jax.ShapeDtypeStruct((2, N, 16), _f32),
    )
    hp4 = tc4(acc3, hp3, dinv, b3[None, :], W4p)

    acc4 = _sc_agg32(hp4, src500, dst500, zeros16)
    tc5 = pl.pallas_call(
        _tc5_body,
        grid=(GRID,),
        in_specs=[_row_spec(32), _split_spec(16), _row_spec(1),
                  _full_spec(1, 32)],
        out_specs=_row_spec(32),
        out_shape=jax.ShapeDtypeStruct((N, 32), _f32),
    )
    out = tc5(acc4, hp4, dinv, b4p[None, :])
    return out[:, :OUT_C]
